# baseline (speedup 1.0000x reference)
import jax
import jax.numpy as jnp
from jax import lax
from jax.experimental import pallas as pl
from jax.experimental.pallas import tpu as pltpu
from jax.experimental.pallas import tpu_sc as plsc

H = 512
W = 512
B = 16
HP = H + 1
MW = 257
TBL = HP * MW
NPIX = B * H * W

NC, NS = 2, 16
NW = NC * NS
PIX_PER_W = NPIX // NW
CHUNK = 4096
NCHUNK = PIX_PER_W // CHUNK
ROWS_PER_CHUNK = CHUNK // W
GRP = W // 16
NSTREAM = CHUNK // 128
MAGIC = 12582912.0


def _warp_body(table_hbm, disp_hbm, lin_hbm, out_hbm,
               lin_v, disp_v, idx_v, x_v, y_v, rows_v, out_v, sem):
    wid = lax.axis_index("s") * NC + lax.axis_index("c")
    b = wid // 2
    half = wid % 2
    tbl_off = b * TBL

    pltpu.sync_copy(lin_hbm, lin_v)

    iota = lax.iota(jnp.int32, 16)
    iota2 = iota * 2

    def chunk_body(c, carry):
        pixbase = b * (H * W) + half * PIX_PER_W + c * CHUNK
        row0 = half * (H // 2) + c * ROWS_PER_CHUNK
        pltpu.sync_copy(disp_hbm.at[pl.ds(pixbase * 2, CHUNK * 2)], disp_v)

        def pass1(r, carry1):
            ax = plsc.load_gather(lin_v, [jnp.full((16,), row0 + r, jnp.int32)])
            base_r = r * W
            for g in range(GRP):
                base = base_r + g * 16
                dx = plsc.load_gather(disp_v, [iota2 + 2 * base])
                dy = plsc.load_gather(disp_v, [iota2 + (2 * base + 1)])
                ay = lin_v[pl.ds(g * 16, 16)]
                x = (0.5 * ((ax - dx) + 1.0)) * jnp.float32(HP - 2)
                y = (0.5 * ((ay - dy) + 1.0)) * jnp.float32(HP - 2)
                x_v[pl.ds(base, 16)] = x
                y_v[pl.ds(base, 16)] = y
                xr = (x + MAGIC) - MAGIC
                yr = (y + MAGIC) - MAGIC
                x0 = jnp.minimum(jnp.maximum(xr, 0.0),
                                 jnp.float32(HP - 1)).astype(jnp.int32)
                y0 = jnp.minimum(jnp.maximum(yr, 0.0),
                                 jnp.float32(HP - 1)).astype(jnp.int32)
                idx_v[r * (W // 128) + g // 8, pl.ds((g % 8) * 16, 16)] = (
                    x0 * MW + jnp.right_shift(y0, 1) + tbl_off)
            return carry1

        lax.fori_loop(0, ROWS_PER_CHUNK, pass1, 0)

        copies = []
        for t in range(NSTREAM):
            copies.append(pltpu.async_copy(
                table_hbm.at[idx_v.at[t]], rows_v.at[t], sem))
        for cp in copies:
            cp.wait()

        def pass2(r, carry2):
            base_r = r * W
            for g in range(GRP):
                base = base_r + g * 16
                x = x_v[pl.ds(base, 16)]
                y = y_v[pl.ds(base, 16)]
                xr = (x + MAGIC) - MAGIC
                yr = (y + MAGIC) - MAGIC
                x0f = jnp.minimum(jnp.maximum(xr, 0.0), jnp.float32(HP - 1))
                y0f = jnp.minimum(jnp.maximum(yr, 0.0), jnp.float32(HP - 1))
                x1f = jnp.minimum(x0f + 1.0, jnp.float32(HP - 1))
                y1f = jnp.minimum(y0f + 1.0, jnp.float32(HP - 1))
                e = jnp.bitwise_and(y0f.astype(jnp.int32), 1)
                tvec = jnp.full((16,), r * (W // 128) + g // 8, jnp.int32)
                ridx = iota + (g % 8) * 16
                i00 = plsc.load_gather(rows_v, [tvec, ridx, e])
                i01 = plsc.load_gather(rows_v, [tvec, ridx, e + 1])
                i10 = plsc.load_gather(rows_v, [tvec, ridx, e + 4])
                i11 = plsc.load_gather(rows_v, [tvec, ridx, e + 5])
                wx0 = x1f - x
                wx1 = x - x0f
                wy0 = y1f - y
                wy1 = y - y0f
                out = (((wx0 * wy0) * i00 + (wx0 * wy1) * i01)
                       + (wx1 * wy0) * i10) + (wx1 * wy1) * i11
                out = jnp.where((xr >= 0.0) & (yr >= 0.0), out, 0.0)
                out_v[pl.ds(base, 16)] = out
            return carry2

        lax.fori_loop(0, ROWS_PER_CHUNK, pass2, 0)
        pltpu.sync_copy(out_v, out_hbm.at[pl.ds(pixbase, CHUNK)])
        return carry

    lax.fori_loop(0, NCHUNK, chunk_body, 0)


def _build_warp():
    mesh = plsc.VectorSubcoreMesh(core_axis_name="c", subcore_axis_name="s",
                                  num_cores=NC, num_subcores=NS)
    return pl.kernel(
        _warp_body,
        out_type=jax.ShapeDtypeStruct((NPIX,), jnp.float32),
        mesh=mesh,
        compiler_params=pltpu.CompilerParams(needs_layout_passes=False,
                                             use_tc_tiling_on_sc=False),
        scratch_types=[
            pltpu.VMEM((W,), jnp.float32),
            pltpu.VMEM((CHUNK * 2,), jnp.float32),
            pltpu.VMEM((NSTREAM, 128), jnp.int32),
            pltpu.VMEM((CHUNK,), jnp.float32),
            pltpu.VMEM((CHUNK,), jnp.float32),
            pltpu.VMEM((NSTREAM, 128, 8), jnp.float32),
            pltpu.VMEM((CHUNK,), jnp.float32),
            pltpu.SemaphoreType.DMA,
        ],
    )


def kernel(Img, DispField):
    imgp = jnp.pad(Img[..., 0], ((0, 0), (0, 2), (0, 4)))
    parts = [imgp[:, 0:HP, d::2][:, :, :MW] for d in range(4)]
    parts += [imgp[:, 1:HP + 1, d::2][:, :, :MW] for d in range(4)]
    table = jnp.stack(parts, axis=-1).reshape(B * TBL, 8)
    dispflat = DispField.reshape(NPIX * 2)
    lin = jnp.linspace(-1.0, 1.0, H).astype(jnp.float32)
    out = _build_warp()(table, dispflat, lin)
    return out.reshape(B, H, W, 1)


# speedup vs baseline: 1.4945x; 1.4945x over previous
import jax
import jax.numpy as jnp
from jax import lax
from jax.experimental import pallas as pl
from jax.experimental.pallas import tpu as pltpu
from jax.experimental.pallas import tpu_sc as plsc

H = 512
W = 512
B = 16
HP = H + 1
MW = 257
TBL = HP * MW
NPIX = B * H * W

NC, NS = 2, 16
NW = NC * NS
PIX_PER_W = NPIX // NW
CHUNK = 4096
NCHUNK = PIX_PER_W // CHUNK
ROWS_PER_CHUNK = CHUNK // W
GRP = W // 16
NSTREAM = CHUNK // 128
MAGIC = 12582912.0


def _warp_body(table_hbm, disp_hbm, lin_hbm, out_hbm,
               lin_v, disp_v, idx_v, x_v, y_v, rows_v, out_v, sem):
    wid = lax.axis_index("s") * NC + lax.axis_index("c")
    b = wid // 2
    half = wid % 2
    tbl_off = b * TBL

    pltpu.sync_copy(lin_hbm, lin_v)

    iota = lax.iota(jnp.int32, 16)

    def chunk_body(c, carry):
        pixbase = b * (H * W) + half * PIX_PER_W + c * CHUNK
        row0 = half * (H // 2) + c * ROWS_PER_CHUNK
        pltpu.sync_copy(disp_hbm.at[b, pl.ds(row0, ROWS_PER_CHUNK)], disp_v)

        def pass1(r, carry1):
            ax = plsc.load_gather(lin_v, [jnp.full((16,), row0 + r, jnp.int32)])
            base_r = r * W
            for g in range(GRP):
                base = base_r + g * 16
                dx = disp_v[r, g // 8, 0, pl.ds((g % 8) * 16, 16)]
                dy = disp_v[r, g // 8, 1, pl.ds((g % 8) * 16, 16)]
                ay = lin_v[pl.ds(g * 16, 16)]
                x = (0.5 * ((ax - dx) + 1.0)) * jnp.float32(HP - 2)
                y = (0.5 * ((ay - dy) + 1.0)) * jnp.float32(HP - 2)
                x_v[pl.ds(base, 16)] = x
                y_v[pl.ds(base, 16)] = y
                xr = (x + MAGIC) - MAGIC
                yr = (y + MAGIC) - MAGIC
                x0 = jnp.minimum(jnp.maximum(xr, 0.0),
                                 jnp.float32(HP - 1)).astype(jnp.int32)
                y0 = jnp.minimum(jnp.maximum(yr, 0.0),
                                 jnp.float32(HP - 1)).astype(jnp.int32)
                idx_v[r * (W // 128) + g // 8, pl.ds((g % 8) * 16, 16)] = (
                    x0 * MW + jnp.right_shift(y0, 1) + tbl_off)
            return carry1

        lax.fori_loop(0, ROWS_PER_CHUNK, pass1, 0)

        copies = []
        for t in range(NSTREAM):
            copies.append(pltpu.async_copy(
                table_hbm.at[idx_v.at[t]], rows_v.at[t], sem))
        for cp in copies:
            cp.wait()

        def pass2(r, carry2):
            base_r = r * W
            for g in range(GRP):
                base = base_r + g * 16
                x = x_v[pl.ds(base, 16)]
                y = y_v[pl.ds(base, 16)]
                xr = (x + MAGIC) - MAGIC
                yr = (y + MAGIC) - MAGIC
                x0f = jnp.minimum(jnp.maximum(xr, 0.0), jnp.float32(HP - 1))
                y0f = jnp.minimum(jnp.maximum(yr, 0.0), jnp.float32(HP - 1))
                x1f = jnp.minimum(x0f + 1.0, jnp.float32(HP - 1))
                y1f = jnp.minimum(y0f + 1.0, jnp.float32(HP - 1))
                e = jnp.bitwise_and(y0f.astype(jnp.int32), 1)
                tvec = jnp.full((16,), r * (W // 128) + g // 8, jnp.int32)
                ridx = iota + (g % 8) * 16
                i00 = plsc.load_gather(rows_v, [tvec, ridx, e])
                i01 = plsc.load_gather(rows_v, [tvec, ridx, e + 1])
                i10 = plsc.load_gather(rows_v, [tvec, ridx, e + 4])
                i11 = plsc.load_gather(rows_v, [tvec, ridx, e + 5])
                wx0 = x1f - x
                wx1 = x - x0f
                wy0 = y1f - y
                wy1 = y - y0f
                out = (((wx0 * wy0) * i00 + (wx0 * wy1) * i01)
                       + (wx1 * wy0) * i10) + (wx1 * wy1) * i11
                out = jnp.where((xr >= 0.0) & (yr >= 0.0), out, 0.0)
                out_v[pl.ds(base, 16)] = out
            return carry2

        lax.fori_loop(0, ROWS_PER_CHUNK, pass2, 0)
        pltpu.sync_copy(out_v, out_hbm.at[pl.ds(pixbase, CHUNK)])
        return carry

    lax.fori_loop(0, NCHUNK, chunk_body, 0)


def _build_warp():
    mesh = plsc.VectorSubcoreMesh(core_axis_name="c", subcore_axis_name="s",
                                  num_cores=NC, num_subcores=NS)
    return pl.kernel(
        _warp_body,
        out_type=jax.ShapeDtypeStruct((NPIX,), jnp.float32),
        mesh=mesh,
        compiler_params=pltpu.CompilerParams(needs_layout_passes=False,
                                             use_tc_tiling_on_sc=False),
        scratch_types=[
            pltpu.VMEM((W,), jnp.float32),
            pltpu.VMEM((ROWS_PER_CHUNK, 4, 2, 128), jnp.float32),
            pltpu.VMEM((NSTREAM, 128), jnp.int32),
            pltpu.VMEM((CHUNK,), jnp.float32),
            pltpu.VMEM((CHUNK,), jnp.float32),
            pltpu.VMEM((NSTREAM, 128, 8), jnp.float32),
            pltpu.VMEM((CHUNK,), jnp.float32),
            pltpu.SemaphoreType.DMA,
        ],
    )


def kernel(Img, DispField):
    imgp = jnp.pad(Img[..., 0], ((0, 0), (0, 2), (0, 4)))
    parts = [imgp[:, 0:HP, d::2][:, :, :MW] for d in range(4)]
    parts += [imgp[:, 1:HP + 1, d::2][:, :, :MW] for d in range(4)]
    table = jnp.stack(parts, axis=-1).reshape(B * TBL, 8)
    dispn = jnp.transpose(DispField.reshape(B, H, 4, 128, 2), (0, 1, 2, 4, 3))
    lin = jnp.linspace(-1.0, 1.0, H).astype(jnp.float32)
    out = _build_warp()(table, dispn, lin)
    return out.reshape(B, H, W, 1)


# speedup vs baseline: 6.9086x; 4.6228x over previous
import jax
import jax.numpy as jnp
from jax import lax
from jax.experimental import pallas as pl
from jax.experimental.pallas import tpu as pltpu
from jax.experimental.pallas import tpu_sc as plsc

H = 512
W = 512
B = 16
HP = H + 1
MW = 257
TBL = HP * MW
NPIX = B * H * W

NC, NS = 2, 16
NW = NC * NS
PIX_PER_W = NPIX // NW
CHUNK = 4096
NCHUNK = PIX_PER_W // CHUNK
ROWS_PER_CHUNK = CHUNK // W
GRP = W // 16
NSTREAM = CHUNK // 128
MAGIC = 12582912.0


def _warp_body(table_hbm, disp_hbm, lin_hbm, out_hbm,
               lin_v, disp_v, idx_v, x_v, y_v, rows_v, out_v, sem):
    wid = lax.axis_index("s") * NC + lax.axis_index("c")
    b = wid // 2
    half = wid % 2
    tbl_off = b * TBL

    pltpu.sync_copy(lin_hbm, lin_v)

    iota = lax.iota(jnp.int32, 16)

    def chunk_body(c, carry):
        pixbase = b * (H * W) + half * PIX_PER_W + c * CHUNK
        row0 = half * (H // 2) + c * ROWS_PER_CHUNK
        pltpu.sync_copy(disp_hbm.at[b, pl.ds(row0, ROWS_PER_CHUNK)], disp_v)

        def pass1(r, carry1):
            ax = plsc.load_gather(lin_v, [jnp.full((16,), row0 + r, jnp.int32)])
            base_r = r * W
            for g in range(GRP):
                base = base_r + g * 16
                dx = disp_v[r, g // 8, 0, pl.ds((g % 8) * 16, 16)]
                dy = disp_v[r, g // 8, 1, pl.ds((g % 8) * 16, 16)]
                ay = lin_v[pl.ds(g * 16, 16)]
                x = (0.5 * ((ax - dx) + 1.0)) * jnp.float32(HP - 2)
                y = (0.5 * ((ay - dy) + 1.0)) * jnp.float32(HP - 2)
                x_v[pl.ds(base, 16)] = x
                y_v[pl.ds(base, 16)] = y
                xr = (x + MAGIC) - MAGIC
                yr = (y + MAGIC) - MAGIC
                x0 = jnp.minimum(jnp.maximum(xr, 0.0),
                                 jnp.float32(HP - 1)).astype(jnp.int32)
                y0 = jnp.minimum(jnp.maximum(yr, 0.0),
                                 jnp.float32(HP - 1)).astype(jnp.int32)
                idx_v[r * (W // 128) + g // 8, pl.ds((g % 8) * 16, 16)] = (
                    x0 * MW + jnp.right_shift(y0, 1) + tbl_off)
            return carry1

        lax.fori_loop(0, ROWS_PER_CHUNK, pass1, 0)

        copies = []
        for t in range(NSTREAM):
            copies.append(pltpu.async_copy(
                table_hbm.at[idx_v.at[t]], rows_v.at[t], sem))
        for cp in copies:
            cp.wait()

        def pass2(r, carry2):
            base_r = r * W
            for g in range(GRP):
                base = base_r + g * 16
                x = x_v[pl.ds(base, 16)]
                y = y_v[pl.ds(base, 16)]
                xr = (x + MAGIC) - MAGIC
                yr = (y + MAGIC) - MAGIC
                x0f = jnp.minimum(jnp.maximum(xr, 0.0), jnp.float32(HP - 1))
                y0f = jnp.minimum(jnp.maximum(yr, 0.0), jnp.float32(HP - 1))
                x1f = jnp.minimum(x0f + 1.0, jnp.float32(HP - 1))
                y1f = jnp.minimum(y0f + 1.0, jnp.float32(HP - 1))
                e = jnp.bitwise_and(y0f.astype(jnp.int32), 1)
                tvec = jnp.full((16,), r * (W // 128) + g // 8, jnp.int32)
                ridx = iota + (g % 8) * 16
                i00 = plsc.load_gather(rows_v, [tvec, ridx, e])
                i01 = plsc.load_gather(rows_v, [tvec, ridx, e + 1])
                i10 = plsc.load_gather(rows_v, [tvec, ridx, e + 4])
                i11 = plsc.load_gather(rows_v, [tvec, ridx, e + 5])
                wx0 = x1f - x
                wx1 = x - x0f
                wy0 = y1f - y
                wy1 = y - y0f
                out = (((wx0 * wy0) * i00 + (wx0 * wy1) * i01)
                       + (wx1 * wy0) * i10) + (wx1 * wy1) * i11
                out = jnp.where((xr >= 0.0) & (yr >= 0.0), out, 0.0)
                out_v[pl.ds(base, 16)] = out
            return carry2

        lax.fori_loop(0, ROWS_PER_CHUNK, pass2, 0)
        pltpu.sync_copy(out_v, out_hbm.at[pl.ds(pixbase, CHUNK)])
        return carry

    lax.fori_loop(0, NCHUNK, chunk_body, 0)


def _build_body(img_hbm, table_hbm, rbuf, obuf):
    wid = lax.axis_index("s") * NC + lax.axis_index("c")
    b = wid // 2
    half = wid % 2
    x0 = half * MW
    ngx = MW - half

    z = jnp.zeros((16,), jnp.float32)
    rbuf[0, pl.ds(512, 16)] = z
    rbuf[1, pl.ds(512, 16)] = z

    iota = lax.iota(jnp.int32, 16)
    m16 = jnp.right_shift(iota, 3)
    d16 = jnp.bitwise_and(iota, 7)
    rowsel = jnp.where(d16 < 4, 0, 1)
    colbase = 2 * m16 + jnp.bitwise_and(d16, 3)

    def gx_body(i, carry):
        x = x0 + i
        gx = b * (HP) + x

        @pl.when(x <= H - 2)
        def _():
            pltpu.sync_copy(img_hbm.at[b, pl.ds(x, 2), :],
                            rbuf.at[:, pl.ds(0, 512)])

        @pl.when(x == H - 1)
        def _():
            pltpu.sync_copy(img_hbm.at[b, pl.ds(x, 1), :],
                            rbuf.at[pl.ds(0, 1), pl.ds(0, 512)])
            for q in range(32):
                rbuf[1, pl.ds(q * 16, 16)] = z

        @pl.when(x == H)
        def _():
            for q in range(32):
                rbuf[0, pl.ds(q * 16, 16)] = z
                rbuf[1, pl.ds(q * 16, 16)] = z

        for gg in range(129):
            vals = plsc.load_gather(rbuf, [rowsel, colbase + 4 * gg])
            plsc.store_scatter(obuf, [m16 + 2 * gg, d16], vals)
        pltpu.sync_copy(obuf.at[pl.ds(0, MW), :],
                        table_hbm.at[pl.ds(gx * MW, MW), :])
        return carry

    lax.fori_loop(0, ngx, gx_body, 0)


def _make_builder():
    mesh = plsc.VectorSubcoreMesh(core_axis_name="c", subcore_axis_name="s",
                                  num_cores=NC, num_subcores=NS)
    return pl.kernel(
        _build_body,
        out_type=jax.ShapeDtypeStruct((B * TBL, 8), jnp.float32),
        mesh=mesh,
        compiler_params=pltpu.CompilerParams(needs_layout_passes=False,
                                             use_tc_tiling_on_sc=False),
        scratch_types=[
            pltpu.VMEM((2, 528), jnp.float32),
            pltpu.VMEM((260, 8), jnp.float32),
        ],
    )


def _build_warp():
    mesh = plsc.VectorSubcoreMesh(core_axis_name="c", subcore_axis_name="s",
                                  num_cores=NC, num_subcores=NS)
    return pl.kernel(
        _warp_body,
        out_type=jax.ShapeDtypeStruct((NPIX,), jnp.float32),
        mesh=mesh,
        compiler_params=pltpu.CompilerParams(needs_layout_passes=False,
                                             use_tc_tiling_on_sc=False),
        scratch_types=[
            pltpu.VMEM((W,), jnp.float32),
            pltpu.VMEM((ROWS_PER_CHUNK, 4, 2, 128), jnp.float32),
            pltpu.VMEM((NSTREAM, 128), jnp.int32),
            pltpu.VMEM((CHUNK,), jnp.float32),
            pltpu.VMEM((CHUNK,), jnp.float32),
            pltpu.VMEM((NSTREAM, 128, 8), jnp.float32),
            pltpu.VMEM((CHUNK,), jnp.float32),
            pltpu.SemaphoreType.DMA,
        ],
    )


def kernel(Img, DispField):
    table = _make_builder()(Img[..., 0])
    dispn = jnp.transpose(DispField.reshape(B, H, 4, 128, 2), (0, 1, 2, 4, 3))
    lin = jnp.linspace(-1.0, 1.0, H).astype(jnp.float32)
    out = _build_warp()(table, dispn, lin)
    return out.reshape(B, H, W, 1)


# speedup vs baseline: 8.2702x; 1.1971x over previous
import jax
import jax.numpy as jnp
from jax import lax
from jax.experimental import pallas as pl
from jax.experimental.pallas import tpu as pltpu
from jax.experimental.pallas import tpu_sc as plsc

H = 512
W = 512
B = 16
HP = H + 1
MW = 257
TBL = HP * MW
NPIX = B * H * W

NC, NS = 2, 16
NW = NC * NS
PIX_PER_W = NPIX // NW
CHUNK = 4096
NCHUNK = PIX_PER_W // CHUNK
ROWS_PER_CHUNK = CHUNK // W
GRP = W // 16
NSTREAM = CHUNK // 128
MAGIC = 12582912.0


def _warp_body(table_hbm, disp_hbm, lin_hbm, out_hbm,
               lin_v, disp_v, idx_v, x_v, y_v, rows_v, out_v, sem):
    wid = lax.axis_index("s") * NC + lax.axis_index("c")
    b = wid // 2
    half = wid % 2
    tbl_off = b * TBL

    pltpu.sync_copy(lin_hbm, lin_v)

    iota = lax.iota(jnp.int32, 16)

    def chunk_body(c, carry):
        pixbase = b * (H * W) + half * PIX_PER_W + c * CHUNK
        row0 = half * (H // 2) + c * ROWS_PER_CHUNK
        pltpu.sync_copy(disp_hbm.at[b, pl.ds(row0, ROWS_PER_CHUNK)], disp_v)

        def pass1(r, carry1):
            ax = plsc.load_gather(lin_v, [jnp.full((16,), row0 + r, jnp.int32)])
            base_r = r * W
            for g in range(GRP):
                base = base_r + g * 16
                dx = disp_v[r, g // 8, 0, pl.ds((g % 8) * 16, 16)]
                dy = disp_v[r, g // 8, 1, pl.ds((g % 8) * 16, 16)]
                ay = lin_v[pl.ds(g * 16, 16)]
                x = (0.5 * ((ax - dx) + 1.0)) * jnp.float32(HP - 2)
                y = (0.5 * ((ay - dy) + 1.0)) * jnp.float32(HP - 2)
                x_v[pl.ds(base, 16)] = x
                y_v[pl.ds(base, 16)] = y
                xr = (x + MAGIC) - MAGIC
                yr = (y + MAGIC) - MAGIC
                x0 = jnp.minimum(jnp.maximum(xr, 0.0),
                                 jnp.float32(HP - 1)).astype(jnp.int32)
                y0 = jnp.minimum(jnp.maximum(yr, 0.0),
                                 jnp.float32(HP - 1)).astype(jnp.int32)
                idx_v[r * (W // 128) + g // 8, pl.ds((g % 8) * 16, 16)] = (
                    x0 * MW + jnp.right_shift(y0, 1) + tbl_off)
                if g % 8 == 7:
                    t = r * (W // 128) + g // 8
                    pltpu.async_copy(
                        table_hbm.at[idx_v.at[t]], rows_v.at[t], sem)
            return carry1

        lax.fori_loop(0, ROWS_PER_CHUNK, pass1, 0)

        for t in range(NSTREAM):
            pltpu.make_async_copy(
                table_hbm.at[idx_v.at[t]], rows_v.at[t], sem).wait()

        def pass2(r, carry2):
            base_r = r * W
            for g in range(GRP):
                base = base_r + g * 16
                x = x_v[pl.ds(base, 16)]
                y = y_v[pl.ds(base, 16)]
                xr = (x + MAGIC) - MAGIC
                yr = (y + MAGIC) - MAGIC
                x0f = jnp.minimum(jnp.maximum(xr, 0.0), jnp.float32(HP - 1))
                y0f = jnp.minimum(jnp.maximum(yr, 0.0), jnp.float32(HP - 1))
                x1f = jnp.minimum(x0f + 1.0, jnp.float32(HP - 1))
                y1f = jnp.minimum(y0f + 1.0, jnp.float32(HP - 1))
                e = jnp.bitwise_and(y0f.astype(jnp.int32), 1)
                tvec = jnp.full((16,), r * (W // 128) + g // 8, jnp.int32)
                ridx = iota + (g % 8) * 16
                i00 = plsc.load_gather(rows_v, [tvec, ridx, e])
                i01 = plsc.load_gather(rows_v, [tvec, ridx, e + 1])
                i10 = plsc.load_gather(rows_v, [tvec, ridx, e + 4])
                i11 = plsc.load_gather(rows_v, [tvec, ridx, e + 5])
                wx0 = x1f - x
                wx1 = x - x0f
                wy0 = y1f - y
                wy1 = y - y0f
                out = (((wx0 * wy0) * i00 + (wx0 * wy1) * i01)
                       + (wx1 * wy0) * i10) + (wx1 * wy1) * i11
                out = jnp.where((xr >= 0.0) & (yr >= 0.0), out, 0.0)
                out_v[pl.ds(base, 16)] = out
            return carry2

        lax.fori_loop(0, ROWS_PER_CHUNK, pass2, 0)
        pltpu.sync_copy(out_v, out_hbm.at[pl.ds(pixbase, CHUNK)])
        return carry

    lax.fori_loop(0, NCHUNK, chunk_body, 0)


def _build_body(img_hbm, table_hbm, rbuf, obuf, osem):
    wid = lax.axis_index("s") * NC + lax.axis_index("c")
    b = wid // 2
    half = wid % 2
    x0 = half * MW
    ngx = MW - half

    z = jnp.zeros((16,), jnp.float32)
    rbuf[0, pl.ds(512, 16)] = z
    rbuf[1, pl.ds(512, 16)] = z

    iota = lax.iota(jnp.int32, 16)
    m16 = jnp.right_shift(iota, 3)
    d16 = jnp.bitwise_and(iota, 7)
    rowsel = jnp.where(d16 < 4, 0, 1)
    colbase = 2 * m16 + jnp.bitwise_and(d16, 3)

    def gx_body(i, carry):
        x = x0 + i
        gx = b * (HP) + x

        @pl.when(x <= H - 2)
        def _():
            pltpu.sync_copy(img_hbm.at[b, pl.ds(x, 2), :],
                            rbuf.at[:, pl.ds(0, 512)])

        @pl.when(x == H - 1)
        def _():
            pltpu.sync_copy(img_hbm.at[b, pl.ds(x, 1), :],
                            rbuf.at[pl.ds(0, 1), pl.ds(0, 512)])
            for q in range(32):
                rbuf[1, pl.ds(q * 16, 16)] = z

        @pl.when(x == H)
        def _():
            for q in range(32):
                rbuf[0, pl.ds(q * 16, 16)] = z
                rbuf[1, pl.ds(q * 16, 16)] = z

        @pl.when(i > 0)
        def _():
            pltpu.make_async_copy(
                obuf.at[pl.ds(0, MW), :],
                table_hbm.at[pl.ds((gx - 1) * MW, MW), :], osem).wait()

        for gg in range(129):
            vals = plsc.load_gather(rbuf, [rowsel, colbase + 4 * gg])
            plsc.store_scatter(obuf, [m16 + 2 * gg, d16], vals)
        pltpu.async_copy(obuf.at[pl.ds(0, MW), :],
                         table_hbm.at[pl.ds(gx * MW, MW), :], osem)
        return carry

    lax.fori_loop(0, ngx, gx_body, 0)
    lastgx = b * HP + x0 + ngx - 1
    pltpu.make_async_copy(obuf.at[pl.ds(0, MW), :],
                          table_hbm.at[pl.ds(lastgx * MW, MW), :], osem).wait()


def _make_builder():
    mesh = plsc.VectorSubcoreMesh(core_axis_name="c", subcore_axis_name="s",
                                  num_cores=NC, num_subcores=NS)
    return pl.kernel(
        _build_body,
        out_type=jax.ShapeDtypeStruct((B * TBL, 8), jnp.float32),
        mesh=mesh,
        compiler_params=pltpu.CompilerParams(needs_layout_passes=False,
                                             use_tc_tiling_on_sc=False),
        scratch_types=[
            pltpu.VMEM((2, 528), jnp.float32),
            pltpu.VMEM((260, 8), jnp.float32),
            pltpu.SemaphoreType.DMA,
        ],
    )


def _build_warp():
    mesh = plsc.VectorSubcoreMesh(core_axis_name="c", subcore_axis_name="s",
                                  num_cores=NC, num_subcores=NS)
    return pl.kernel(
        _warp_body,
        out_type=jax.ShapeDtypeStruct((NPIX,), jnp.float32),
        mesh=mesh,
        compiler_params=pltpu.CompilerParams(needs_layout_passes=False,
                                             use_tc_tiling_on_sc=False),
        scratch_types=[
            pltpu.VMEM((W,), jnp.float32),
            pltpu.VMEM((ROWS_PER_CHUNK, 4, 2, 128), jnp.float32),
            pltpu.VMEM((NSTREAM, 128), jnp.int32),
            pltpu.VMEM((CHUNK,), jnp.float32),
            pltpu.VMEM((CHUNK,), jnp.float32),
            pltpu.VMEM((NSTREAM, 128, 8), jnp.float32),
            pltpu.VMEM((CHUNK,), jnp.float32),
            pltpu.SemaphoreType.DMA,
        ],
    )


def kernel(Img, DispField):
    table = _make_builder()(Img[..., 0])
    dispn = jnp.transpose(DispField.reshape(B, H, 4, 128, 2), (0, 1, 2, 4, 3))
    lin = jnp.linspace(-1.0, 1.0, H).astype(jnp.float32)
    out = _build_warp()(table, dispn, lin)
    return out.reshape(B, H, W, 1)


# speedup vs baseline: 10.4457x; 1.2631x over previous
import jax
import jax.numpy as jnp
from jax import lax
from jax.experimental import pallas as pl
from jax.experimental.pallas import tpu as pltpu
from jax.experimental.pallas import tpu_sc as plsc

H = 512
W = 512
B = 16
HP = H + 1
MW = 257
TBL = HP * MW
NPIX = B * H * W

NC, NS = 2, 16
NW = NC * NS
PIX_PER_W = NPIX // NW
CHUNK = 4096
NCHUNK = PIX_PER_W // CHUNK
ROWS_PER_CHUNK = CHUNK // W
GRP = W // 16
NSTREAM = CHUNK // 128
MAGIC = 12582912.0


def _warp_body(table_hbm, disp_hbm, lin_hbm, out_hbm,
               lin_v, disp_v, idx_v, x_v, y_v, rows_v, out_v, sem):
    wid = lax.axis_index("s") * NC + lax.axis_index("c")
    b = wid // 2
    half = wid % 2
    tbl_off = b * TBL

    pltpu.sync_copy(lin_hbm, lin_v)

    iota = lax.iota(jnp.int32, 16)

    def chunk_body(c, carry):
        pixbase = b * (H * W) + half * PIX_PER_W + c * CHUNK
        row0 = half * (H // 2) + c * ROWS_PER_CHUNK
        pltpu.sync_copy(disp_hbm.at[b, pl.ds(row0, ROWS_PER_CHUNK)], disp_v)

        def pass1(r, carry1):
            ax = plsc.load_gather(lin_v, [jnp.full((16,), row0 + r, jnp.int32)])
            base_r = r * W
            for g in range(GRP):
                base = base_r + g * 16
                dx = disp_v[r, g // 8, 0, pl.ds((g % 8) * 16, 16)]
                dy = disp_v[r, g // 8, 1, pl.ds((g % 8) * 16, 16)]
                ay = lin_v[pl.ds(g * 16, 16)]
                x = (0.5 * ((ax - dx) + 1.0)) * jnp.float32(HP - 2)
                y = (0.5 * ((ay - dy) + 1.0)) * jnp.float32(HP - 2)
                x_v[pl.ds(base, 16)] = x
                y_v[pl.ds(base, 16)] = y
                xr = (x + MAGIC) - MAGIC
                yr = (y + MAGIC) - MAGIC
                x0 = jnp.minimum(jnp.maximum(xr, 0.0),
                                 jnp.float32(HP - 1)).astype(jnp.int32)
                y0 = jnp.minimum(jnp.maximum(yr, 0.0),
                                 jnp.float32(HP - 1)).astype(jnp.int32)
                idx_v[r * (W // 128) + g // 8, pl.ds((g % 8) * 16, 16)] = (
                    x0 * MW + jnp.right_shift(y0, 1) + tbl_off)
                if g % 8 == 7:
                    t = r * (W // 128) + g // 8
                    pltpu.async_copy(
                        table_hbm.at[idx_v.at[t]], rows_v.at[t], sem)
            return carry1

        lax.fori_loop(0, ROWS_PER_CHUNK, pass1, 0)

        for t in range(NSTREAM):
            pltpu.make_async_copy(
                table_hbm.at[idx_v.at[t]], rows_v.at[t], sem).wait()

        def pass2(r, carry2):
            base_r = r * W
            for g in range(GRP):
                base = base_r + g * 16
                x = x_v[pl.ds(base, 16)]
                y = y_v[pl.ds(base, 16)]
                xr = (x + MAGIC) - MAGIC
                yr = (y + MAGIC) - MAGIC
                x0f = jnp.minimum(jnp.maximum(xr, 0.0), jnp.float32(HP - 1))
                y0f = jnp.minimum(jnp.maximum(yr, 0.0), jnp.float32(HP - 1))
                x1f = jnp.minimum(x0f + 1.0, jnp.float32(HP - 1))
                y1f = jnp.minimum(y0f + 1.0, jnp.float32(HP - 1))
                e = jnp.bitwise_and(y0f.astype(jnp.int32), 1)
                tvec = jnp.full((16,), r * (W // 128) + g // 8, jnp.int32)
                ridx = iota + (g % 8) * 16
                i00 = plsc.load_gather(rows_v, [tvec, ridx, e])
                i01 = plsc.load_gather(rows_v, [tvec, ridx, e + 1])
                i10 = plsc.load_gather(rows_v, [tvec, ridx, e + 4])
                i11 = plsc.load_gather(rows_v, [tvec, ridx, e + 5])
                wx0 = x1f - x
                wx1 = x - x0f
                wy0 = y1f - y
                wy1 = y - y0f
                out = (((wx0 * wy0) * i00 + (wx0 * wy1) * i01)
                       + (wx1 * wy0) * i10) + (wx1 * wy1) * i11
                out = jnp.where((xr >= 0.0) & (yr >= 0.0), out, 0.0)
                out_v[pl.ds(base, 16)] = out
            return carry2

        lax.fori_loop(0, ROWS_PER_CHUNK, pass2, 0)
        pltpu.sync_copy(out_v, out_hbm.at[pl.ds(pixbase, CHUNK)])
        return carry

    lax.fori_loop(0, NCHUNK, chunk_body, 0)


def _build_body(img_hbm, table_hbm, rbuf, obuf, osem, isem):
    wid = lax.axis_index("s") * NC + lax.axis_index("c")
    b = wid // 2
    half = wid % 2
    x0 = half * MW
    ngx = MW - half

    z = jnp.zeros((16,), jnp.float32)
    for p in range(2):
        rbuf[p, 0, pl.ds(512, 16)] = z
        rbuf[p, 1, pl.ds(512, 16)] = z
        for q in range(33):
            rbuf[p, 2, pl.ds(q * 16, 16)] = z

    iota = lax.iota(jnp.int32, 16)
    m16 = jnp.right_shift(iota, 3)
    d16 = jnp.bitwise_and(iota, 7)
    colbase = 2 * m16 + jnp.bitwise_and(d16, 3)

    def src_of(x):
        return jnp.minimum(x, H - 2)

    pltpu.async_copy(img_hbm.at[b, pl.ds(src_of(x0), 2), :],
                     rbuf.at[0, pl.ds(0, 2), pl.ds(0, 512)], isem)

    def gx_body(i, carry):
        x = x0 + i
        gx = b * (HP) + x
        p = jnp.bitwise_and(i, 1)

        pltpu.make_async_copy(
            img_hbm.at[b, pl.ds(src_of(x), 2), :],
            rbuf.at[p, pl.ds(0, 2), pl.ds(0, 512)], isem).wait()

        @pl.when(i + 1 < ngx)
        def _():
            pltpu.async_copy(
                img_hbm.at[b, pl.ds(src_of(x + 1), 2), :],
                rbuf.at[1 - p, pl.ds(0, 2), pl.ds(0, 512)], isem)

        arow = jnp.where(x == H - 1, 1, jnp.where(x == H, 2, 0))
        brow = jnp.where(x <= H - 2, 1, 2)
        rowsel = jnp.where(d16 < 4, arow, brow)
        pvec = jnp.full((16,), p, jnp.int32)

        @pl.when(i > 0)
        def _():
            pltpu.make_async_copy(
                obuf.at[pl.ds(0, MW), :],
                table_hbm.at[pl.ds((gx - 1) * MW, MW), :], osem).wait()

        for gg in range(129):
            vals = plsc.load_gather(rbuf, [pvec, rowsel, colbase + 4 * gg])
            plsc.store_scatter(obuf, [m16 + 2 * gg, d16], vals)
        pltpu.async_copy(obuf.at[pl.ds(0, MW), :],
                         table_hbm.at[pl.ds(gx * MW, MW), :], osem)
        return carry

    lax.fori_loop(0, ngx, gx_body, 0)
    lastgx = b * HP + x0 + ngx - 1
    pltpu.make_async_copy(obuf.at[pl.ds(0, MW), :],
                          table_hbm.at[pl.ds(lastgx * MW, MW), :], osem).wait()


def _make_builder():
    mesh = plsc.VectorSubcoreMesh(core_axis_name="c", subcore_axis_name="s",
                                  num_cores=NC, num_subcores=NS)
    return pl.kernel(
        _build_body,
        out_type=jax.ShapeDtypeStruct((B * TBL, 8), jnp.float32),
        mesh=mesh,
        compiler_params=pltpu.CompilerParams(needs_layout_passes=False,
                                             use_tc_tiling_on_sc=False),
        scratch_types=[
            pltpu.VMEM((2, 3, 528), jnp.float32),
            pltpu.VMEM((260, 8), jnp.float32),
            pltpu.SemaphoreType.DMA,
            pltpu.SemaphoreType.DMA,
        ],
    )


def _build_warp():
    mesh = plsc.VectorSubcoreMesh(core_axis_name="c", subcore_axis_name="s",
                                  num_cores=NC, num_subcores=NS)
    return pl.kernel(
        _warp_body,
        out_type=jax.ShapeDtypeStruct((NPIX,), jnp.float32),
        mesh=mesh,
        compiler_params=pltpu.CompilerParams(needs_layout_passes=False,
                                             use_tc_tiling_on_sc=False),
        scratch_types=[
            pltpu.VMEM((W,), jnp.float32),
            pltpu.VMEM((ROWS_PER_CHUNK, 4, 2, 128), jnp.float32),
            pltpu.VMEM((NSTREAM, 128), jnp.int32),
            pltpu.VMEM((CHUNK,), jnp.float32),
            pltpu.VMEM((CHUNK,), jnp.float32),
            pltpu.VMEM((NSTREAM, 128, 8), jnp.float32),
            pltpu.VMEM((CHUNK,), jnp.float32),
            pltpu.SemaphoreType.DMA,
        ],
    )


def kernel(Img, DispField):
    table = _make_builder()(Img[..., 0])
    dispn = jnp.transpose(DispField.reshape(B, H, 4, 128, 2), (0, 1, 2, 4, 3))
    lin = jnp.linspace(-1.0, 1.0, H).astype(jnp.float32)
    out = _build_warp()(table, dispn, lin)
    return out.reshape(B, H, W, 1)
